# baseline (speedup 1.0000x reference)
import functools

import jax
import jax.numpy as jnp
from jax import lax
from jax.experimental import pallas as pl
from jax.experimental.pallas import tpu as pltpu
from jax.experimental.pallas import tpu_sc as plsc

_B, _T, _E = 4096, 200, 32
_NW = 32
_BS = 128
_NBT = _B // _BS
_NTT = _T // 8

_mesh = plsc.VectorSubcoreMesh(core_axis_name="c", subcore_axis_name="s")


@functools.partial(
    pl.kernel,
    out_type=jax.ShapeDtypeStruct((_T, _E // 8, _NBT, 8, _BS), jnp.float32),
    mesh=_mesh,
    scratch_types=[
        pltpu.VMEM((_NTT, 8, _BS), jnp.int32),
        pltpu.VMEM((2, _BS, _E), jnp.float32),
        pltpu.VMEM((2, _E // 8, 8, _BS + 1), jnp.float32),
        pltpu.SemaphoreType.DMA,
        pltpu.SemaphoreType.DMA,
        pltpu.SemaphoreType.DMA,
        pltpu.SemaphoreType.DMA,
    ],
    compiler_params=pltpu.CompilerParams(
        use_tc_tiling_on_sc=False, needs_layout_passes=False
    ),
)
def _sc_gather(idx_hbm, table_hbm, out_hbm, idx_v, gbuf, tbuf, g0, g1, o0, o1):
    wid = lax.axis_index("s") * 2 + lax.axis_index("c")
    pltpu.sync_copy(idx_hbm.at[pl.ds(0, _NTT), wid], idx_v)

    lanes = lax.iota(jnp.int32, 16)
    et_vec = [(lanes + e0) // 8 for e0 in (0, 16)]
    es_vec = [(lanes + e0) % 8 for e0 in (0, 16)]
    zero_vec = lanes * 0

    def start_gather(t, slot, sem):
        pltpu.async_copy(
            table_hbm.at[idx_v.at[t // 8, t % 8]], gbuf.at[slot], sem
        )

    def wait_gather(slot, sem):
        pltpu.make_async_copy(
            table_hbm.at[idx_v.at[0, 0]], gbuf.at[slot], sem
        ).wait()

    def transpose(slot):
        dst = tbuf.at[slot]
        for b in range(_BS):
            bvec = zero_vec + b
            for h in (0, 1):
                vals = gbuf[slot, b, pl.ds(16 * h, 16)]
                plsc.store_scatter(dst, [et_vec[h], es_vec[h], bvec], vals)

    def start_store(t, slot, sem):
        pltpu.async_copy(
            tbuf.at[slot, pl.ds(0, _E // 8), pl.ds(0, 8), pl.ds(0, _BS)],
            out_hbm.at[t, pl.ds(0, _E // 8), wid],
            sem,
        )

    def wait_store(slot, sem):
        pltpu.make_async_copy(
            tbuf.at[slot, pl.ds(0, _E // 8), pl.ds(0, 8), pl.ds(0, _BS)],
            out_hbm.at[0, pl.ds(0, _E // 8), wid],
            sem,
        ).wait()

    start_gather(0, 0, g0)

    _H = _T // 2

    @pl.loop(0, _H)
    def _pair(i):
        ta, tb = 2 * i, 2 * i + 1
        start_gather(tb, 1, g1)
        wait_gather(0, g0)
        transpose(0)
        start_store(ta, 0, o0)

        @pl.when(i < _H - 1)
        def _():
            start_gather(tb + 1, 0, g0)

        wait_gather(1, g1)
        transpose(1)
        start_store(tb, 1, o1)
        wait_store(0, o0)
        wait_store(1, o1)


def kernel(input_, table):
    idx4 = input_.reshape(_NBT, _BS, _NTT, 8).transpose(2, 0, 3, 1)
    out5 = _sc_gather(idx4, table)
    return out5.transpose(2, 4, 0, 1, 3).reshape(_B, _T, _E)


# speedup vs baseline: 1.0200x; 1.0200x over previous
import functools

import jax
import jax.numpy as jnp
from jax import lax
from jax.experimental import pallas as pl
from jax.experimental.pallas import tpu as pltpu
from jax.experimental.pallas import tpu_sc as plsc

_B, _T, _E = 4096, 200, 32
_NW = 32
_BS = 128
_NBT = _B // _BS
_NTT = _T // 8

_mesh = plsc.VectorSubcoreMesh(core_axis_name="c", subcore_axis_name="s")


@functools.partial(
    pl.kernel,
    out_type=jax.ShapeDtypeStruct((_T, _E // 8, _NBT, 8, _BS), jnp.float32),
    mesh=_mesh,
    scratch_types=[
        pltpu.VMEM((_NTT, 8, _BS), jnp.int32),
        pltpu.VMEM((2, _BS, _E), jnp.float32),
        pltpu.VMEM((2, _E // 8, 8, _BS + 1), jnp.float32),
        pltpu.SemaphoreType.DMA,
        pltpu.SemaphoreType.DMA,
        pltpu.SemaphoreType.DMA,
        pltpu.SemaphoreType.DMA,
    ],
    compiler_params=pltpu.CompilerParams(
        use_tc_tiling_on_sc=False, needs_layout_passes=False
    ),
)
def _sc_gather(idx_hbm, table_hbm, out_hbm, idx_v, gbuf, tbuf, g0, g1, o0, o1):
    wid = lax.axis_index("s") * 2 + lax.axis_index("c")
    pltpu.sync_copy(idx_hbm.at[pl.ds(0, _NTT), wid], idx_v)

    lanes = lax.iota(jnp.int32, 16)
    et_vec = [(lanes + e0) // 8 for e0 in (0, 16)]
    es_vec = [(lanes + e0) % 8 for e0 in (0, 16)]
    zero_vec = lanes * 0

    def start_gather(t, slot, sem):
        pltpu.async_copy(
            table_hbm.at[idx_v.at[t // 8, t % 8]], gbuf.at[slot], sem
        )

    def wait_gather(slot, sem):
        pltpu.make_async_copy(
            table_hbm.at[idx_v.at[0, 0]], gbuf.at[slot], sem
        ).wait()

    def transpose(slot):
        dst = tbuf.at[slot]
        for b in range(_BS):
            bvec = zero_vec + b
            for h in (0, 1):
                vals = gbuf[slot, b, pl.ds(16 * h, 16)]
                plsc.store_scatter(dst, [et_vec[h], es_vec[h], bvec], vals)

    def start_store(t, slot, sem):
        pltpu.async_copy(
            tbuf.at[slot, pl.ds(0, _E // 8), pl.ds(0, 8), pl.ds(0, _BS)],
            out_hbm.at[t, pl.ds(0, _E // 8), wid],
            sem,
        )

    def wait_store(slot, sem):
        pltpu.make_async_copy(
            tbuf.at[slot, pl.ds(0, _E // 8), pl.ds(0, 8), pl.ds(0, _BS)],
            out_hbm.at[0, pl.ds(0, _E // 8), wid],
            sem,
        ).wait()

    start_gather(0, 0, g0)

    _H = _T // 2

    @pl.loop(0, _H)
    def _pair(i):
        ta, tb = 2 * i, 2 * i + 1
        start_gather(tb, 1, g1)
        wait_gather(0, g0)

        @pl.when(i > 0)
        def _():
            wait_store(0, o0)

        transpose(0)
        start_store(ta, 0, o0)

        @pl.when(i < _H - 1)
        def _():
            start_gather(tb + 1, 0, g0)

        wait_gather(1, g1)

        @pl.when(i > 0)
        def _():
            wait_store(1, o1)

        transpose(1)
        start_store(tb, 1, o1)

    wait_store(0, o0)
    wait_store(1, o1)


def kernel(input_, table):
    idx4 = input_.reshape(_NBT, _BS, _NTT, 8).transpose(2, 0, 3, 1)
    out5 = _sc_gather(idx4, table)
    return out5.transpose(2, 4, 0, 1, 3).reshape(_B, _T, _E)
